# baseline (speedup 1.0000x reference)
import jax
import jax.numpy as jnp
from jax import lax
from jax.experimental import pallas as pl
from jax.experimental.pallas import tpu as pltpu
from jax.experimental.pallas import tpu_sc as plsc

VOCAB = 100000
NGRAM_VOCAB = 1000000
DIM = 64
BATCH = 4096
NGRAMS = 20

NC, NS, L = 2, 16, 16
NW = NC * NS
BW = BATCH // NW
DV = DIM // L
NBUF = 3

WBLK = 256
NFULL = NGRAM_VOCAB // WBLK
TAIL = NGRAM_VOCAB - NFULL * WBLK
TMAX = (NFULL + NW - 1) // NW
LEFT = NFULL - (TMAX - 1) * NW


def _transpose_body(net, out, inbuf, outbuf, tailbuf,
                    gsem0, gsem1, osem0, osem1):
    cid = lax.axis_index("c")
    sid = lax.axis_index("s")
    wid = sid * NC + cid
    iota = jax.lax.iota(jnp.int32, L)
    last_ok = wid < LEFT

    def in_cp(t, j, sem):
        c0 = (wid + NW * t) * WBLK
        return pltpu.async_copy(net.at[:, pl.ds(c0, WBLK)], inbuf.at[j], sem)

    def transpose_block(src, j, nrows):
        @pl.loop(0, nrows)
        def _(r):
            for c in range(8):
                v = plsc.load_gather(
                    src,
                    [(c % 4) * L + iota,
                     jnp.full((L,), 0, jnp.int32) + 2 * r + c // 4])
                outbuf[j, r, pl.ds(c * L, L)] = v

    def out_cp(t, j, sem):
        p0 = (wid + NW * t) * (WBLK // 2)
        return pltpu.async_copy(outbuf.at[j], out.at[pl.ds(p0, WBLK // 2), :],
                                sem)

    @pl.loop(0, (TMAX - 1) // 2)
    def _(o):
        t0 = 2 * o
        ci0 = in_cp(t0, 0, gsem0)
        ci1 = in_cp(t0 + 1, 1, gsem1)
        ci0.wait()
        transpose_block(inbuf.at[0], 0, WBLK // 2)
        co0 = out_cp(t0, 0, osem0)
        ci1.wait()
        transpose_block(inbuf.at[1], 1, WBLK // 2)
        co1 = out_cp(t0 + 1, 1, osem1)
        co0.wait()
        co1.wait()

    @pl.when(last_ok)
    def _():
        in_cp(TMAX - 1, 0, gsem0).wait()
        transpose_block(inbuf.at[0], 0, WBLK // 2)
        out_cp(TMAX - 1, 0, osem0).wait()

    @pl.when(wid == NW - 1)
    def _():
        c0 = NFULL * WBLK
        pltpu.async_copy(net.at[:, pl.ds(c0, TAIL)], tailbuf, gsem0).wait()
        transpose_block(tailbuf, 0, TAIL // 2)
        pltpu.async_copy(outbuf.at[0].at[pl.ds(0, TAIL // 2), :],
                         out.at[pl.ds(c0 // 2, TAIL // 2), :], osem0).wait()


def _score_body(we, ne, cwi, cnt, xwi, xnt, out,
                widx, tnv, wbuf, rows, cacc, xacc, outv,
                shacc, sem_w, gs0, gs1, gs2, ss0, ss1, ss2):
    cid = lax.axis_index("c")
    sid = lax.axis_index("s")
    wid = sid * NC + cid
    base = wid * BW
    iota = jax.lax.iota(jnp.int32, L)
    gsems = (gs0, gs1, gs2)
    ssems = (ss0, ss1, ss2)

    for k, (wsrc, nsrc) in enumerate(((cwi, cnt), (xwi, xnt))):
        pltpu.sync_copy(wsrc.at[pl.ds(base, BW)], widx.at[k])
        pltpu.sync_copy(nsrc.at[:, pl.ds(base, BW)], tnv.at[k])

    for k in range(2):
        rowbase = (sid * 2 + k) * BW
        for i in range(BW // L):
            widx[2 + k, pl.ds(i * L, L)] = rowbase + i * L + iota

    for k in range(2):
        pltpu.async_copy(we.at[widx.at[k]], wbuf, sem_w).wait()
        pltpu.sync_copy(wbuf, shacc.at[widx.at[2 + k]])

    steps = [(k, g) for g in range(NGRAMS) for k in range(2)]
    gather_cp = [None] * NBUF
    sct_cp = [None] * NBUF

    def fire_gather(step_i):
        k, g = steps[step_i]
        j = step_i % NBUF
        gather_cp[j] = pltpu.async_copy(
            ne.at[tnv.at[k, g]], rows.at[j], gsems[j])

    for i in range(NBUF):
        fire_gather(i)

    for i in range(len(steps)):
        k, g = steps[i]
        j = i % NBUF
        gather_cp[j].wait()
        sct_cp[j] = pltpu.async_copy(
            rows.at[j], shacc.at[widx.at[2 + k]], ssems[j], add=True)
        if i + NBUF < len(steps):
            sct_cp[j].wait()
            sct_cp[j] = None
            fire_gather(i + NBUF)

    for j in range(NBUF):
        if sct_cp[j] is not None:
            sct_cp[j].wait()

    pltpu.sync_copy(shacc.at[pl.ds((sid * 2) * BW, BW), :], cacc)
    pltpu.sync_copy(shacc.at[pl.ds((sid * 2 + 1) * BW, BW), :], xacc)

    for b0 in range(BW // L):
        ridx = b0 * L + iota

        @pl.loop(0, DIM, init_carry=jnp.zeros((L,), jnp.float32))
        def s(d, s):
            col = jnp.full((L,), 0, jnp.int32) + d
            c = plsc.load_gather(cacc, [ridx, col])
            x = plsc.load_gather(xacc, [ridx, col])
            return s + c * x

        outv[pl.ds(b0 * L, L)] = s

    pltpu.sync_copy(outv, out.at[pl.ds(base, BW)])


@jax.jit
def _run(we, ne_t, cwi, cnt, xwi, xnt):
    mesh = plsc.VectorSubcoreMesh(core_axis_name="c", subcore_axis_name="s",
                                  num_cores=NC, num_subcores=NS)
    transpose = pl.kernel(
        _transpose_body,
        out_type=jax.ShapeDtypeStruct((NGRAM_VOCAB // 2, 2 * DIM),
                                      jnp.float32),
        mesh=mesh,
        compiler_params=pltpu.CompilerParams(
            needs_layout_passes=False, use_tc_tiling_on_sc=True),
        scratch_types=[
            pltpu.VMEM((2, DIM, WBLK), jnp.float32),
            pltpu.VMEM((2, WBLK // 2, 2 * DIM), jnp.float32),
            pltpu.VMEM((DIM, TAIL), jnp.float32),
            pltpu.SemaphoreType.DMA,
            pltpu.SemaphoreType.DMA,
            pltpu.SemaphoreType.DMA,
            pltpu.SemaphoreType.DMA,
        ],
    )
    ne_lin = transpose(ne_t).reshape(NGRAM_VOCAB, DIM)

    score = pl.kernel(
        _score_body,
        out_type=jax.ShapeDtypeStruct((BATCH,), jnp.float32),
        mesh=mesh,
        compiler_params=pltpu.CompilerParams(
            needs_layout_passes=False, use_tc_tiling_on_sc=False),
        scratch_types=[
            pltpu.VMEM((4, BW), jnp.int32),
            pltpu.VMEM((2, NGRAMS, BW), jnp.int32),
            pltpu.VMEM((BW, DIM), jnp.float32),
            pltpu.VMEM((NBUF, BW, DIM), jnp.float32),
            pltpu.VMEM((BW, DIM), jnp.float32),
            pltpu.VMEM((BW, DIM), jnp.float32),
            pltpu.VMEM((BW,), jnp.float32),
            pltpu.VMEM_SHARED((NS * 2 * BW, DIM), jnp.float32),
            pltpu.SemaphoreType.DMA,
            pltpu.SemaphoreType.DMA,
            pltpu.SemaphoreType.DMA,
            pltpu.SemaphoreType.DMA,
            pltpu.SemaphoreType.DMA,
            pltpu.SemaphoreType.DMA,
            pltpu.SemaphoreType.DMA,
        ],
    )
    return score(we, ne_lin, cwi, cnt, xwi, xnt)


def kernel(word_embeddings, ngram_embeddings, center_word_idx,
           center_ngram_idxs, context_word_idx, context_ngram_idxs):
    return _run(
        word_embeddings,
        ngram_embeddings.T,
        center_word_idx.astype(jnp.int32),
        center_ngram_idxs.astype(jnp.int32).T,
        context_word_idx.astype(jnp.int32),
        context_ngram_idxs.astype(jnp.int32).T)
